# baseline (speedup 1.0000x reference)
import functools

import jax
import jax.numpy as jnp
from jax import lax
from jax.experimental import pallas as pl
from jax.experimental.pallas import tpu as pltpu
from jax.experimental.pallas import tpu_sc as plsc

_EPS = 1e-12
_LANES = 16


def _xlane_sum(x):
    iot = lax.iota(jnp.int32, _LANES)
    for k in (1, 2, 4, 8):
        x = x + x.at[iot ^ k].get(mode="promise_in_bounds")
    return x


def _build_sc_call(N, S, D, C):
    NC, NS = 2, 16
    NW = NC * NS
    per_w = N // NW
    chunks = per_w // C
    assert chunks % 2 == 0
    nj = D // _LANES

    mesh = plsc.VectorSubcoreMesh(core_axis_name="c", subcore_axis_name="s")

    @functools.partial(
        pl.kernel,
        mesh=mesh,
        out_type=jax.ShapeDtypeStruct((N, D), jnp.float32),
        scratch_types=[
            pltpu.VMEM((per_w,), jnp.int32),
            pltpu.VMEM((per_w,), jnp.int32),
            pltpu.VMEM((C,), jnp.int32),
            pltpu.VMEM((C,), jnp.int32),
            pltpu.VMEM((C, D), jnp.float32),
            pltpu.VMEM((C, D), jnp.float32),
            pltpu.VMEM((C, D // 2), jnp.int32),
            pltpu.VMEM((C, D // 2), jnp.int32),
            pltpu.VMEM((D,), jnp.int32),
            pltpu.VMEM((2, D), jnp.float32),
            pltpu.SemaphoreType.DMA,
            pltpu.SemaphoreType.DMA,
            pltpu.SemaphoreType.DMA,
            pltpu.SemaphoreType.DMA,
            pltpu.SemaphoreType.DMA,
            pltpu.SemaphoreType.DMA,
        ],
    )
    def sc_call(word_hbm, comb_hbm, ids_hbm, tt_hbm, gb_hbm,
                out_hbm, ids_v, tt_v, idx_c0, idx_c1,
                w0, w1, a0, a1, gb_v, x_v,
                sem_w0, sem_w1, sem_a0, sem_a1, sem_o0, sem_o1):
        sid = lax.axis_index("s")
        wid = sid * NC + lax.axis_index("c")
        idx_cs = (idx_c0, idx_c1)
        ws = (w0, w1)
        aas = (a0, a1)
        sem_w = (sem_w0, sem_w1)
        sem_a = (sem_a0, sem_a1)
        sem_o = (sem_o0, sem_o1)

        base0 = wid * per_w
        pltpu.sync_copy(gb_hbm, gb_v)
        pltpu.sync_copy(ids_hbm.at[pl.ds(base0, per_w)], ids_v)
        pltpu.sync_copy(tt_hbm.at[pl.ds(base0, per_w)], tt_v)

        def start_gathers(ch, p):
            off = ch * C
            s_base = lax.rem(base0 + off, S)
            for k in range(C // _LANES):
                sl = pl.ds(_LANES * k, _LANES)
                ttv = tt_v[pl.ds(off + _LANES * k, _LANES)]
                svec = s_base + _LANES * k + lax.iota(jnp.int32, _LANES)
                idx_cs[p][sl] = ttv + 2 * svec
            pltpu.make_async_copy(word_hbm.at[ids_v.at[pl.ds(off, C)]],
                                  ws[p], sem_w[p]).start()
            pltpu.make_async_copy(comb_hbm.at[idx_cs[p]], aas[p],
                                  sem_a[p]).start()

        def out_copy(ch, p):
            return pltpu.make_async_copy(
                ws[p], out_hbm.at[pl.ds(base0 + ch * C, C)], sem_o[p])

        start_gathers(0, 0)

        def make_pair_range(p):
            def row_body(i, rc):
                rows = (2 * i, 2 * i + 1)
                zero = jnp.zeros((_LANES,), jnp.float32)
                init = (zero,) * 8

                @plsc.parallel_loop(0, nj // 2, unroll=2, carry=init)
                def stats(jj, acc):
                    acc = list(acc)
                    slp = pl.ds(_LANES * jj, _LANES)
                    for q, r in enumerate(rows):
                        av = aas[p][r, slp]
                        alo = lax.bitcast_convert_type(
                            lax.shift_left(av, 16), jnp.float32)
                        ahi = lax.bitcast_convert_type(
                            av & jnp.int32(-65536), jnp.float32)
                        for half, ab in ((0, alo), (1, ahi)):
                            sl = pl.ds(_LANES * (2 * jj + half), _LANES)
                            x = ws[p][r, sl] + ab
                            x_v[q, sl] = x
                            k = 4 * half + 2 * q
                            acc[k] = acc[k] + x
                            acc[k + 1] = acc[k + 1] + x * x
                    return tuple(acc)

                meanv = []
                yv = []
                for q in range(2):
                    s1 = _xlane_sum(stats[2 * q] + stats[4 + 2 * q])
                    s2 = _xlane_sum(stats[2 * q + 1] + stats[4 + 2 * q + 1])
                    m = s1 * (1.0 / D)
                    vv = s2 * (1.0 / D) - m * m + _EPS
                    ii = lax.bitcast_convert_type(vv, jnp.int32)
                    ii = 0x5F3759DF - lax.shift_right_arithmetic(ii, 1)
                    y = lax.bitcast_convert_type(ii, jnp.float32)
                    for _ in range(2):
                        y = y * (1.5 - 0.5 * vv * y * y)
                    meanv.append(m)
                    yv.append(y)

                @plsc.parallel_loop(0, nj, unroll=4)
                def normalize(j):
                    sl = pl.ds(_LANES * j, _LANES)
                    gbv = gb_v[sl]
                    g = lax.bitcast_convert_type(
                        gbv & jnp.int32(-65536), jnp.float32)
                    b = lax.bitcast_convert_type(
                        lax.shift_left(gbv, 16), jnp.float32)
                    for q, r in enumerate(rows):
                        xn = (x_v[q, sl] - meanv[q]) * yv[q]
                        ws[p][r, sl] = xn * g + b

                return rc

            return row_body

        def pair_body(g, carry):
            for p in (0, 1):
                ch = 2 * g + p
                pltpu.make_async_copy(word_hbm.at[ids_v.at[pl.ds(0, C)]],
                                      ws[p], sem_w[p]).wait()
                pltpu.make_async_copy(comb_hbm.at[idx_cs[p]], aas[p],
                                      sem_a[p]).wait()
                row_body = make_pair_range(p)
                lax.fori_loop(0, C // 4, row_body, 0)

                @pl.when(ch >= 1)
                def _(p=p, ch=ch):
                    out_copy(ch - 1, 1 - p).wait()

                @pl.when(ch + 1 < chunks)
                def _(p=p, ch=ch):
                    start_gathers(ch + 1, 1 - p)

                lax.fori_loop(C // 4, C // 2, row_body, 0)
                out_copy(ch, p).start()
            return carry

        lax.fori_loop(0, chunks // 2, pair_body, 0)
        out_copy(chunks - 1, (chunks - 1) % 2).wait()

    return sc_call


def kernel(input_ids, token_type_ids, word_embeddings, position_embeddings,
           token_type_embeddings, ln_gamma, ln_beta):
    B, S = input_ids.shape
    V, D = word_embeddings.shape
    N = B * S
    comb = (position_embeddings[:S, None, :]
            + token_type_embeddings[None, :, :]).reshape(2 * S, D)
    cr = comb.reshape(2 * S, D // 32, 2, _LANES)
    lo = lax.bitcast_convert_type(
        cr[:, :, 0, :].astype(jnp.bfloat16), jnp.uint16).astype(jnp.uint32)
    hi = lax.bitcast_convert_type(
        cr[:, :, 1, :].astype(jnp.bfloat16), jnp.uint16).astype(jnp.uint32)
    comb_packed = ((hi << 16) | lo).astype(jnp.int32).reshape(2 * S, D // 2)
    ids = input_ids.reshape(N)
    tt = token_type_ids.reshape(N)
    g16 = lax.bitcast_convert_type(
        ln_gamma.astype(jnp.bfloat16), jnp.uint16).astype(jnp.uint32)
    b16 = lax.bitcast_convert_type(
        ln_beta.astype(jnp.bfloat16), jnp.uint16).astype(jnp.uint32)
    gb = ((g16 << 16) | b16).astype(jnp.int32)
    sc_call = _build_sc_call(N, S, D, C=32)
    out = sc_call(word_embeddings, comb_packed, ids, tt, gb)
    return out.reshape(B, S, D)


# speedup vs baseline: 1.1092x; 1.1092x over previous
import functools

import jax
import jax.numpy as jnp
from jax import lax
from jax.experimental import pallas as pl
from jax.experimental.pallas import tpu as pltpu
from jax.experimental.pallas import tpu_sc as plsc

_EPS = 1e-12
_LANES = 16


def _xlane_sum(x):
    iot = lax.iota(jnp.int32, _LANES)
    for k in (1, 2, 4, 8):
        x = x + x.at[iot ^ k].get(mode="promise_in_bounds")
    return x


def _build_sc_call(N, S, D, C):
    NC, NS = 2, 16
    NW = NC * NS
    per_w = N // NW
    chunks = per_w // C
    assert chunks % 2 == 0
    nj = D // _LANES

    mesh = plsc.VectorSubcoreMesh(core_axis_name="c", subcore_axis_name="s")

    @functools.partial(
        pl.kernel,
        mesh=mesh,
        out_type=jax.ShapeDtypeStruct((N, D), jnp.float32),
        scratch_types=[
            pltpu.VMEM((per_w,), jnp.int32),
            pltpu.VMEM((per_w,), jnp.int32),
            pltpu.VMEM((C,), jnp.int32),
            pltpu.VMEM((C,), jnp.int32),
            pltpu.VMEM((C, D), jnp.float32),
            pltpu.VMEM((C, D), jnp.float32),
            pltpu.VMEM((C, D), jnp.float32),
            pltpu.VMEM((C, D), jnp.float32),
            pltpu.VMEM((D,), jnp.int32),
            pltpu.VMEM((2, D), jnp.float32),
            pltpu.SemaphoreType.DMA,
            pltpu.SemaphoreType.DMA,
            pltpu.SemaphoreType.DMA,
            pltpu.SemaphoreType.DMA,
            pltpu.SemaphoreType.DMA,
            pltpu.SemaphoreType.DMA,
        ],
    )
    def sc_call(word_hbm, comb_hbm, ids_hbm, tt_hbm, gb_hbm,
                out_hbm, ids_v, tt_v, idx_c0, idx_c1,
                w0, w1, a0, a1, gb_v, x_v,
                sem_w0, sem_w1, sem_a0, sem_a1, sem_o0, sem_o1):
        sid = lax.axis_index("s")
        wid = sid * NC + lax.axis_index("c")
        idx_cs = (idx_c0, idx_c1)
        ws = (w0, w1)
        aas = (a0, a1)
        sem_w = (sem_w0, sem_w1)
        sem_a = (sem_a0, sem_a1)
        sem_o = (sem_o0, sem_o1)

        base0 = wid * per_w
        pltpu.sync_copy(gb_hbm, gb_v)
        pltpu.sync_copy(ids_hbm.at[pl.ds(base0, per_w)], ids_v)
        pltpu.sync_copy(tt_hbm.at[pl.ds(base0, per_w)], tt_v)

        def start_gathers(ch, p):
            off = ch * C
            s_base = lax.rem(base0 + off, S)
            for k in range(C // _LANES):
                sl = pl.ds(_LANES * k, _LANES)
                ttv = tt_v[pl.ds(off + _LANES * k, _LANES)]
                svec = s_base + _LANES * k + lax.iota(jnp.int32, _LANES)
                idx_cs[p][sl] = ttv + 2 * svec
            pltpu.make_async_copy(word_hbm.at[ids_v.at[pl.ds(off, C)]],
                                  ws[p], sem_w[p]).start()
            pltpu.make_async_copy(comb_hbm.at[idx_cs[p]], aas[p],
                                  sem_a[p]).start()

        def out_copy(ch, p):
            return pltpu.make_async_copy(
                aas[p], out_hbm.at[pl.ds(base0 + ch * C, C)], sem_o[p])

        start_gathers(0, 0)

        def make_pair_range(p):
            def row_body(i, rc):
                rows = (2 * i, 2 * i + 1)
                zero = jnp.zeros((_LANES,), jnp.float32)
                init = (zero,) * 8

                @plsc.parallel_loop(0, nj, step=2, unroll=2, carry=init)
                def stats(j, acc):
                    acc = list(acc)
                    for half in range(2):
                        sl = pl.ds(_LANES * (j + half), _LANES)
                        for q, r in enumerate(rows):
                            x = ws[p][r, sl] + aas[p][r, sl]
                            x_v[q, sl] = x
                            k = 4 * half + 2 * q
                            acc[k] = acc[k] + x
                            acc[k + 1] = acc[k + 1] + x * x
                    return tuple(acc)

                meanv = []
                yv = []
                for q in range(2):
                    s1 = _xlane_sum(stats[2 * q] + stats[4 + 2 * q])
                    s2 = _xlane_sum(stats[2 * q + 1] + stats[4 + 2 * q + 1])
                    m = s1 * (1.0 / D)
                    vv = s2 * (1.0 / D) - m * m + _EPS
                    ii = lax.bitcast_convert_type(vv, jnp.int32)
                    ii = 0x5F3759DF - lax.shift_right_arithmetic(ii, 1)
                    y = lax.bitcast_convert_type(ii, jnp.float32)
                    for _ in range(2):
                        y = y * (1.5 - 0.5 * vv * y * y)
                    meanv.append(m)
                    yv.append(y)

                @plsc.parallel_loop(0, nj, unroll=4)
                def normalize(j):
                    sl = pl.ds(_LANES * j, _LANES)
                    gbv = gb_v[sl]
                    g = lax.bitcast_convert_type(
                        gbv & jnp.int32(-65536), jnp.float32)
                    b = lax.bitcast_convert_type(
                        lax.shift_left(gbv, 16), jnp.float32)
                    for q, r in enumerate(rows):
                        xn = (x_v[q, sl] - meanv[q]) * yv[q]
                        aas[p][r, sl] = xn * g + b

                return rc

            return row_body

        def pair_body(g, carry):
            for p in (0, 1):
                ch = 2 * g + p
                pltpu.make_async_copy(word_hbm.at[ids_v.at[pl.ds(0, C)]],
                                      ws[p], sem_w[p]).wait()
                pltpu.make_async_copy(comb_hbm.at[idx_cs[p]], aas[p],
                                      sem_a[p]).wait()
                row_body = make_pair_range(p)
                lax.fori_loop(0, C // 4, row_body, 0)

                @pl.when(ch >= 1)
                def _(p=p, ch=ch):
                    out_copy(ch - 1, 1 - p).wait()

                @pl.when(ch + 1 < chunks)
                def _(p=p, ch=ch):
                    start_gathers(ch + 1, 1 - p)

                lax.fori_loop(C // 4, C // 2, row_body, 0)
                out_copy(ch, p).start()
            return carry

        lax.fori_loop(0, chunks // 2, pair_body, 0)
        out_copy(chunks - 1, (chunks - 1) % 2).wait()

    return sc_call


def kernel(input_ids, token_type_ids, word_embeddings, position_embeddings,
           token_type_embeddings, ln_gamma, ln_beta):
    B, S = input_ids.shape
    V, D = word_embeddings.shape
    N = B * S
    comb = (position_embeddings[:S, None, :]
            + token_type_embeddings[None, :, :]).reshape(2 * S, D)
    ids = input_ids.reshape(N)
    tt = token_type_ids.reshape(N)
    g16 = lax.bitcast_convert_type(
        ln_gamma.astype(jnp.bfloat16), jnp.uint16).astype(jnp.uint32)
    b16 = lax.bitcast_convert_type(
        ln_beta.astype(jnp.bfloat16), jnp.uint16).astype(jnp.uint32)
    gb = ((g16 << 16) | b16).astype(jnp.int32)
    sc_call = _build_sc_call(N, S, D, C=32)
    out = sc_call(word_embeddings, comb, ids, tt, gb)
    return out.reshape(B, S, D)


# speedup vs baseline: 1.1329x; 1.0214x over previous
import functools

import jax
import jax.numpy as jnp
from jax import lax
from jax.experimental import pallas as pl
from jax.experimental.pallas import tpu as pltpu
from jax.experimental.pallas import tpu_sc as plsc

_EPS = 1e-12
_LANES = 16


def _xlane_sum(x):
    iot = lax.iota(jnp.int32, _LANES)
    for k in (1, 2, 4, 8):
        x = x + x.at[iot ^ k].get(mode="promise_in_bounds")
    return x


def _build_sc_call(N, S, D, C):
    NC, NS = 2, 16
    NW = NC * NS
    per_w = N // NW
    chunks = per_w // C
    assert chunks % 2 == 0
    nj = D // _LANES

    mesh = plsc.VectorSubcoreMesh(core_axis_name="c", subcore_axis_name="s")

    @functools.partial(
        pl.kernel,
        mesh=mesh,
        out_type=jax.ShapeDtypeStruct((N, D), jnp.float32),
        scratch_types=[
            pltpu.VMEM((per_w,), jnp.int32),
            pltpu.VMEM((per_w,), jnp.int32),
            pltpu.VMEM((C,), jnp.int32),
            pltpu.VMEM((C,), jnp.int32),
            pltpu.VMEM((C, D), jnp.float32),
            pltpu.VMEM((C, D), jnp.float32),
            pltpu.VMEM((C, D), jnp.float32),
            pltpu.VMEM((C, D), jnp.float32),
            pltpu.VMEM((D,), jnp.int32),
            pltpu.VMEM((2, D), jnp.float32),
            pltpu.SemaphoreType.DMA,
            pltpu.SemaphoreType.DMA,
            pltpu.SemaphoreType.DMA,
            pltpu.SemaphoreType.DMA,
            pltpu.SemaphoreType.DMA,
            pltpu.SemaphoreType.DMA,
        ],
    )
    def sc_call(word_hbm, comb_hbm, ids_hbm, tt_hbm, gb_hbm,
                out_hbm, ids_v, tt_v, idx_c0, idx_c1,
                w0, w1, a0, a1, gb_v, x_v,
                sem_w0, sem_w1, sem_a0, sem_a1, sem_o0, sem_o1):
        sid = lax.axis_index("s")
        wid = sid * NC + lax.axis_index("c")
        idx_cs = (idx_c0, idx_c1)
        ws = (w0, w1)
        aas = (a0, a1)
        sem_w = (sem_w0, sem_w1)
        sem_a = (sem_a0, sem_a1)
        sem_o = (sem_o0, sem_o1)

        base0 = wid * per_w
        pltpu.sync_copy(gb_hbm, gb_v)
        pltpu.sync_copy(ids_hbm.at[pl.ds(base0, per_w)], ids_v)
        pltpu.sync_copy(tt_hbm.at[pl.ds(base0, per_w)], tt_v)

        def start_gathers(ch, p):
            off = ch * C
            s_base = lax.rem(base0 + off, S)
            for k in range(C // _LANES):
                sl = pl.ds(_LANES * k, _LANES)
                ttv = tt_v[pl.ds(off + _LANES * k, _LANES)]
                svec = s_base + _LANES * k + lax.iota(jnp.int32, _LANES)
                idx_cs[p][sl] = ttv + 2 * svec
            pltpu.make_async_copy(word_hbm.at[ids_v.at[pl.ds(off, C)]],
                                  ws[p], sem_w[p]).start()
            pltpu.make_async_copy(comb_hbm.at[idx_cs[p]], aas[p],
                                  sem_a[p]).start()

        def out_copy(ch, p):
            return pltpu.make_async_copy(
                aas[p], out_hbm.at[pl.ds(base0 + ch * C, C)], sem_o[p])

        start_gathers(0, 0)

        def make_pair_range(p):
            def row_body(i, rc):
                rows = (2 * i, 2 * i + 1)
                zero = jnp.zeros((_LANES,), jnp.float32)
                init = (zero,) * 8

                @plsc.parallel_loop(0, nj, step=2, unroll=2, carry=init)
                def stats(j, acc):
                    acc = list(acc)
                    for half in range(2):
                        sl = pl.ds(_LANES * (j + half), _LANES)
                        for q, r in enumerate(rows):
                            x = ws[p][r, sl] + aas[p][r, sl]
                            x_v[q, sl] = x
                            k = 4 * half + 2 * q
                            acc[k] = acc[k] + x
                            acc[k + 1] = acc[k + 1] + x * x
                    return tuple(acc)

                meanv = []
                yv = []
                for q in range(2):
                    s1 = _xlane_sum(stats[2 * q] + stats[4 + 2 * q])
                    s2 = _xlane_sum(stats[2 * q + 1] + stats[4 + 2 * q + 1])
                    m = s1 * (1.0 / D)
                    vv = s2 * (1.0 / D) - m * m + _EPS
                    ii = lax.bitcast_convert_type(vv, jnp.int32)
                    ii = 0x5F3759DF - lax.shift_right_arithmetic(ii, 1)
                    y = lax.bitcast_convert_type(ii, jnp.float32)
                    for _ in range(2):
                        y = y * (1.5 - 0.5 * vv * y * y)
                    meanv.append(m)
                    yv.append(y)

                @plsc.parallel_loop(0, nj, unroll=4)
                def normalize(j):
                    sl = pl.ds(_LANES * j, _LANES)
                    gbv = gb_v[sl]
                    g = lax.bitcast_convert_type(
                        gbv & jnp.int32(-65536), jnp.float32)
                    b = lax.bitcast_convert_type(
                        lax.shift_left(gbv, 16), jnp.float32)
                    for q, r in enumerate(rows):
                        t = yv[q] * g
                        x = x_v[q, sl]
                        aas[p][r, sl] = (x - meanv[q]) * t + b

                return rc

            return row_body

        def pair_body(g, carry):
            for p in (0, 1):
                ch = 2 * g + p
                pltpu.make_async_copy(word_hbm.at[ids_v.at[pl.ds(0, C)]],
                                      ws[p], sem_w[p]).wait()
                pltpu.make_async_copy(comb_hbm.at[idx_cs[p]], aas[p],
                                      sem_a[p]).wait()
                row_body = make_pair_range(p)
                lax.fori_loop(0, C // 4, row_body, 0)

                @pl.when(ch >= 1)
                def _(p=p, ch=ch):
                    out_copy(ch - 1, 1 - p).wait()

                @pl.when(ch + 1 < chunks)
                def _(p=p, ch=ch):
                    start_gathers(ch + 1, 1 - p)

                lax.fori_loop(C // 4, C // 2, row_body, 0)
                out_copy(ch, p).start()
            return carry

        lax.fori_loop(0, chunks // 2, pair_body, 0)
        out_copy(chunks - 1, (chunks - 1) % 2).wait()

    return sc_call


def kernel(input_ids, token_type_ids, word_embeddings, position_embeddings,
           token_type_embeddings, ln_gamma, ln_beta):
    B, S = input_ids.shape
    V, D = word_embeddings.shape
    N = B * S
    comb = (position_embeddings[:S, None, :]
            + token_type_embeddings[None, :, :]).reshape(2 * S, D)
    ids = input_ids.reshape(N)
    tt = token_type_ids.reshape(N)
    g16 = lax.bitcast_convert_type(
        ln_gamma.astype(jnp.bfloat16), jnp.uint16).astype(jnp.uint32)
    b16 = lax.bitcast_convert_type(
        ln_beta.astype(jnp.bfloat16), jnp.uint16).astype(jnp.uint32)
    gb = ((g16 << 16) | b16).astype(jnp.int32)
    sc_call = _build_sc_call(N, S, D, C=32)
    out = sc_call(word_embeddings, comb, ids, tt, gb)
    return out.reshape(B, S, D)
